# baseline (speedup 1.0000x reference)
import jax
import jax.numpy as jnp
from jax.experimental import pallas as pl

_TOK_BLOCK = 2048

_INV_SQRT2 = 0.7071067811865476


def _gelu_exact(x):
    return 0.5 * x * (1.0 + jax.lax.erf(x * _INV_SQRT2))


def _dot(a, b):
    return jax.lax.dot_general(
        a.astype(jnp.bfloat16), b,
        dimension_numbers=(((1,), (0,)), ((), ())),
        preferred_element_type=jnp.float32,
    )


def _fused_kernel(x_ref, pm_ref, wl_ref, w1_ref, w2_ref, w3_ref,
                  out_ref, mask_ref, curr_ref):
    x0 = x_ref[...]
    h = _gelu_exact(_dot(x0, wl_ref[...]))
    h = _gelu_exact(_dot(h, w1_ref[...]))
    h = _gelu_exact(_dot(h, w2_ref[...]))
    logit = jax.lax.dot_general(
        w3_ref[...], h.astype(jnp.bfloat16),
        dimension_numbers=(((1,), (1,)), ((), ())),
        preferred_element_type=jnp.float32,
    )[0:1, :]
    curr = jax.nn.sigmoid(logit) * pm_ref[0, :, :]
    keep = (curr > 0.5).astype(jnp.float32)
    curr_m = keep + 1e-10
    curr_ref[0, :, :] = curr_m
    mask_ref[0, :, :] = curr_m.astype(jnp.int32)
    out_ref[...] = x0 * jnp.transpose(curr_m)


def kernel(input_feature, attention_mask, prev_m, W_L, W1, W2, W3):
    B, S, D = input_feature.shape
    N = B * S
    T = _TOK_BLOCK
    grid = (N // T,)

    x = input_feature.reshape(N, D)
    pm = prev_m.reshape(N // T, 1, T)
    wl = W_L.T.astype(jnp.bfloat16)
    w1 = W1.T.astype(jnp.bfloat16)
    w2 = W2.T.astype(jnp.bfloat16)
    w3 = jnp.pad(W3, ((0, 7), (0, 0))).astype(jnp.bfloat16)

    out, mask, curr_m = pl.pallas_call(
        _fused_kernel,
        grid=grid,
        in_specs=[
            pl.BlockSpec((T, D), lambda i: (i, 0)),
            pl.BlockSpec((1, 1, T), lambda i: (i, 0, 0)),
            pl.BlockSpec(wl.shape, lambda i: (0, 0)),
            pl.BlockSpec(w1.shape, lambda i: (0, 0)),
            pl.BlockSpec(w2.shape, lambda i: (0, 0)),
            pl.BlockSpec(w3.shape, lambda i: (0, 0)),
        ],
        out_specs=[
            pl.BlockSpec((T, D), lambda i: (i, 0)),
            pl.BlockSpec((1, 1, T), lambda i: (i, 0, 0)),
            pl.BlockSpec((1, 1, T), lambda i: (i, 0, 0)),
        ],
        out_shape=[
            jax.ShapeDtypeStruct((N, D), jnp.float32),
            jax.ShapeDtypeStruct((N // T, 1, T), jnp.int32),
            jax.ShapeDtypeStruct((N // T, 1, T), jnp.float32),
        ],
    )(x, pm, wl, w1, w2, w3)

    return (out.reshape(B, S, D), mask.reshape(B, S), curr_m.reshape(B, S))


# speedup vs baseline: 1.0379x; 1.0379x over previous
import jax
import jax.numpy as jnp
from jax.experimental import pallas as pl

_TOK_BLOCK = 4096

_INV_SQRT2 = 0.7071067811865476


def _gelu_exact(x):
    return 0.5 * x * (1.0 + jax.lax.erf(x * _INV_SQRT2))


def _dot(a, b):
    return jax.lax.dot_general(
        a.astype(jnp.bfloat16), b,
        dimension_numbers=(((1,), (0,)), ((), ())),
        preferred_element_type=jnp.float32,
    )


def _fused_kernel(x_ref, pm_ref, wl_ref, w1_ref, w2_ref, w3_ref,
                  out_ref, mask_ref, curr_ref):
    x0 = x_ref[...]
    h = _gelu_exact(_dot(x0, wl_ref[...]))
    h = _gelu_exact(_dot(h, w1_ref[...]))
    h = _gelu_exact(_dot(h, w2_ref[...]))
    logit = jax.lax.dot_general(
        w3_ref[...], h.astype(jnp.bfloat16),
        dimension_numbers=(((1,), (1,)), ((), ())),
        preferred_element_type=jnp.float32,
    )[0:1, :]
    curr = jax.nn.sigmoid(logit) * pm_ref[0, :, :]
    keep = (curr > 0.5).astype(jnp.float32)
    curr_m = keep + 1e-10
    curr_ref[0, :, :] = curr_m
    mask_ref[0, :, :] = curr_m.astype(jnp.int32)
    out_ref[...] = x0 * jnp.transpose(curr_m)


def kernel(input_feature, attention_mask, prev_m, W_L, W1, W2, W3):
    B, S, D = input_feature.shape
    N = B * S
    T = _TOK_BLOCK
    grid = (N // T,)

    x = input_feature.reshape(N, D)
    pm = prev_m.reshape(N // T, 1, T)
    wl = W_L.T.astype(jnp.bfloat16)
    w1 = W1.T.astype(jnp.bfloat16)
    w2 = W2.T.astype(jnp.bfloat16)
    w3 = jnp.pad(W3, ((0, 7), (0, 0))).astype(jnp.bfloat16)

    out, mask, curr_m = pl.pallas_call(
        _fused_kernel,
        grid=grid,
        in_specs=[
            pl.BlockSpec((T, D), lambda i: (i, 0)),
            pl.BlockSpec((1, 1, T), lambda i: (i, 0, 0)),
            pl.BlockSpec(wl.shape, lambda i: (0, 0)),
            pl.BlockSpec(w1.shape, lambda i: (0, 0)),
            pl.BlockSpec(w2.shape, lambda i: (0, 0)),
            pl.BlockSpec(w3.shape, lambda i: (0, 0)),
        ],
        out_specs=[
            pl.BlockSpec((T, D), lambda i: (i, 0)),
            pl.BlockSpec((1, 1, T), lambda i: (i, 0, 0)),
            pl.BlockSpec((1, 1, T), lambda i: (i, 0, 0)),
        ],
        out_shape=[
            jax.ShapeDtypeStruct((N, D), jnp.float32),
            jax.ShapeDtypeStruct((N // T, 1, T), jnp.int32),
            jax.ShapeDtypeStruct((N // T, 1, T), jnp.float32),
        ],
    )(x, pm, wl, w1, w2, w3)

    return (out.reshape(B, S, D), mask.reshape(B, S), curr_m.reshape(B, S))


# speedup vs baseline: 1.1362x; 1.0948x over previous
import jax
import jax.numpy as jnp
from jax.experimental import pallas as pl

_TOK_BLOCK = 4096

_INV_SQRT2 = 0.7071067811865476


def _gelu_exact(x):
    return 0.5 * x * (1.0 + jax.lax.erf(x * _INV_SQRT2))


def _dot_nk(a, b):
    return jax.lax.dot_general(
        a.astype(jnp.bfloat16), b.astype(jnp.bfloat16),
        dimension_numbers=(((1,), (1,)), ((), ())),
        preferred_element_type=jnp.float32,
    )


def _fused_kernel(x_ref, pm_ref, wl_ref, w1_ref, w2_ref, w3_ref,
                  out_ref, mask_ref, curr_ref):
    x0 = x_ref[0]
    h = _gelu_exact(_dot_nk(x0, wl_ref[...]))
    h = _gelu_exact(_dot_nk(h, w1_ref[...]))
    h = _gelu_exact(_dot_nk(h, w2_ref[...]))
    logit = _dot_nk(w3_ref[...], h)
    curr = jax.nn.sigmoid(logit) * pm_ref[0, :, :]
    keep = (curr > 0.5).astype(jnp.float32)
    curr_m = keep + 1e-10
    curr_ref[0, :, :] = curr_m
    mask_ref[0, :, :] = curr_m.astype(jnp.int32)
    out_ref[0] = x0 * jnp.transpose(curr_m)


def kernel(input_feature, attention_mask, prev_m, W_L, W1, W2, W3):
    B, S, D = input_feature.shape
    T = _TOK_BLOCK
    SB = S // T
    grid = (B * SB,)

    pm = prev_m.reshape(B * SB, 1, T)

    out, mask, curr_m = pl.pallas_call(
        _fused_kernel,
        grid=grid,
        in_specs=[
            pl.BlockSpec((1, T, D), lambda i: (i // SB, i % SB, 0)),
            pl.BlockSpec((1, 1, T), lambda i: (i, 0, 0)),
            pl.BlockSpec(W_L.shape, lambda i: (0, 0)),
            pl.BlockSpec(W1.shape, lambda i: (0, 0)),
            pl.BlockSpec(W2.shape, lambda i: (0, 0)),
            pl.BlockSpec(W3.shape, lambda i: (0, 0)),
        ],
        out_specs=[
            pl.BlockSpec((1, T, D), lambda i: (i // SB, i % SB, 0)),
            pl.BlockSpec((1, 1, T), lambda i: (i, 0, 0)),
            pl.BlockSpec((1, 1, T), lambda i: (i, 0, 0)),
        ],
        out_shape=[
            jax.ShapeDtypeStruct((B, S, D), jnp.float32),
            jax.ShapeDtypeStruct((B * SB, 1, T), jnp.int32),
            jax.ShapeDtypeStruct((B * SB, 1, T), jnp.float32),
        ],
    )(input_feature, pm, W_L, W1, W2, W3)

    return (out, mask.reshape(B, S), curr_m.reshape(B, S))
